# baseline (speedup 1.0000x reference)
import functools

import jax
import jax.numpy as jnp
from jax import lax
from jax.experimental import pallas as pl
from jax.experimental.pallas import tpu as pltpu

_CompilerParams = getattr(pltpu, "CompilerParams", None) or getattr(pltpu, "TPUCompilerParams")
_ANY_SPACE = getattr(pl, "ANY", None) or pltpu.TPUMemorySpace.ANY

_VMEM_LIMIT = 64 * 1024 * 1024


def _edge_encode_kernel(ea_ref, xs_ref, xr_ref, u_ref,
                        we_ref, ws_ref, wr_ref, wu_ref, b_ref,
                        ehT_ref, ehTa_ref, esum_ref):
    c = jnp.dot(wu_ref[...], u_ref[...], preferred_element_type=jnp.float32) + b_ref[...]
    y = (jnp.dot(we_ref[...], ea_ref[...], preferred_element_type=jnp.float32)
         + jnp.dot(ws_ref[...], xs_ref[...], preferred_element_type=jnp.float32)
         + jnp.dot(wr_ref[...], xr_ref[...], preferred_element_type=jnp.float32)
         + c)
    ehT = jnp.maximum(y, 0.0)
    ehT_ref[...] = ehT
    te = ehT.shape[1]
    ehTa_ref[...] = jnp.concatenate(
        [ehT.astype(jnp.bfloat16),
         jnp.ones((1, te), jnp.bfloat16),
         jnp.zeros((7, te), jnp.bfloat16)], axis=0)

    @pl.when(pl.program_id(0) == 0)
    def _():
        esum_ref[...] = jnp.zeros_like(esum_ref)

    esum_ref[...] += jnp.sum(ehT, axis=1, keepdims=True)


def _node_kernel(xc_ref, mk_ref, rcol_ref, rrow_ref, srow_ref, ehTa_ref,
                 xrT_ref, xsT_ref, u_ref,
                 wagg_ref, wx_ref, wu_ref, b_ref,
                 nrT_ref, nsT_ref, corr_ref, nsum2_ref,
                 rcb):
    i = pl.program_id(0)

    @pl.when(i == 0)
    def _():
        rcb[...] = jnp.broadcast_to(rcol_ref[...], rcb.shape)

    c = jnp.dot(wu_ref[...], u_ref[...], preferred_element_type=jnp.float32) + b_ref[...]

    ct = jnp.transpose(c)
    ctile = jnp.concatenate([ct] * (mk_ref.shape[1] // ct.shape[1]), axis=1)
    baseN = jnp.maximum(
        jnp.dot(xc_ref[...], mk_ref[...], preferred_element_type=jnp.float32)
        + ctile, 0.0)

    @pl.when(i == 0)
    def _():
        nsum2_ref[...] = jnp.zeros_like(nsum2_ref)
        corr_ref[...] = jnp.zeros_like(corr_ref)

    nsum2_ref[...] += jnp.sum(baseN, axis=0, keepdims=True)

    rs = jnp.concatenate([rrow_ref[...], srow_ref[...]], axis=1)
    rcbv = rcb[...]
    nchunk = rs.shape[1] // 128
    mask = jnp.concatenate(
        [(rcbv == rs[:, k * 128:(k + 1) * 128]).astype(jnp.bfloat16)
         for k in range(nchunk)], axis=1)
    aggT2 = jnp.dot(ehTa_ref[...], mask, preferred_element_type=jnp.float32)

    te = rrow_ref.shape[1]
    aggrT = aggT2[:64, :te]
    multT = aggT2[64:65, :te]
    aggsT = aggT2[:64, te:]

    base_r = jnp.dot(wx_ref[...], xrT_ref[...], preferred_element_type=jnp.float32) + c
    base_s = jnp.dot(wx_ref[...], xsT_ref[...], preferred_element_type=jnp.float32) + c
    nrT = jnp.maximum(
        base_r + jnp.dot(wagg_ref[...], aggrT, preferred_element_type=jnp.float32), 0.0)
    nsT = jnp.maximum(
        base_s + jnp.dot(wagg_ref[...], aggsT, preferred_element_type=jnp.float32), 0.0)
    nrT_ref[...] = nrT
    nsT_ref[...] = nsT

    delta = (nrT - jnp.maximum(base_r, 0.0)) / multT
    corr_ref[...] += jnp.sum(delta, axis=1, keepdims=True)


def _decode_kernel(ehT_ref, nsT_ref, nrT_ref, xrT_ref, xsT_ref,
                   esum_ref, nsum2_ref, fk_ref, corr_ref, u_ref,
                   gwe_ref, gwn_ref, gwu_ref, gb_ref,
                   w1e_ref, w1s_ref, w1r_ref, w1u_ref, b1_ref,
                   w2_ref, b2_ref, out_ref, *, out_dim, inv_e, inv_n):
    e_mean = esum_ref[...] * inv_e
    nsum = lax.dot_general(fk_ref[...], nsum2_ref[...], (((0,), (1,)), ((), ())),
                           preferred_element_type=jnp.float32)
    n_mean = (nsum + corr_ref[...]) * inv_n
    uy = (jnp.dot(gwe_ref[...], e_mean, preferred_element_type=jnp.float32)
          + jnp.dot(gwn_ref[...], n_mean, preferred_element_type=jnp.float32)
          + jnp.dot(gwu_ref[...], u_ref[...], preferred_element_type=jnp.float32)
          + gb_ref[...])
    uhT = jnp.maximum(uy, 0.0)

    cu = jnp.dot(w1u_ref[...], uhT, preferred_element_type=jnp.float32) + b1_ref[...]
    h = (jnp.dot(w1e_ref[...], ehT_ref[...], preferred_element_type=jnp.float32)
         + jnp.dot(w1s_ref[...], nsT_ref[...], preferred_element_type=jnp.float32)
         + jnp.dot(w1r_ref[...], nrT_ref[...], preferred_element_type=jnp.float32)
         + cu)
    h = jnp.maximum(h, 0.0)
    o = jnp.dot(w2_ref[...], h, preferred_element_type=jnp.float32) + b2_ref[...]
    o1 = o[:out_dim, :]
    o2 = o[out_dim:, :]
    xr_row = xrT_ref[2:3, :]
    xs_row = xsT_ref[2:3, :]
    out_ref[...] = o1 * (xr_row - o2 * xs_row)


def _tile(dim, tile):
    t = min(tile, dim)
    if dim % t != 0 or t % 128 != 0:
        t = dim
    return t


def kernel(eb_we, eb_ws, eb_wr, eb_wu, eb_b,
           nb_wagg, nb_wx, nb_wu, nb_b,
           gb_we, gb_wn, gb_wu, gb_b,
           dec1_we, dec1_ws, dec1_wr, dec1_wu, dec1_b,
           dec2_w, dec2_b,
           x, edge_index, edge_attr, u):
    sind, rind = edge_index[0], edge_index[1]
    N, Fn = x.shape
    E, Fe = edge_attr.shape
    GH = u.shape[1]
    EH2 = eb_b.shape[1]
    NH2 = nb_b.shape[1]
    GH2 = gb_b.shape[1]
    OUT2 = dec2_b.shape[1]
    OUT = OUT2 // 2
    EH2A = EH2 + 8

    g = 128 // Fn if (N % (128 // Fn) == 0 and 128 % Fn == 0) else 1
    xc = x.reshape(N // g, g * Fn)
    xcflat = xc.reshape(-1)
    fidx = jnp.arange(Fn, dtype=jnp.int32)[None, :]
    xs = xcflat[edge_index[0].astype(jnp.int32)[:, None] * Fn + fidx]
    xr = xcflat[edge_index[1].astype(jnp.int32)[:, None] * Fn + fidx]
    eaT = edge_attr.T
    xsT = xs.T
    xrT = xr.T
    uT = u.T
    mk = jnp.kron(jnp.eye(g, dtype=x.dtype), nb_wx)
    fk = jnp.tile(jnp.eye(NH2, dtype=x.dtype), (g, 1))
    rind32 = rind.astype(jnp.int32)
    sind32 = sind.astype(jnp.int32)
    rrow = rind32.reshape(1, E)
    srow = sind32.reshape(1, E)
    rcol = rind32.reshape(E, 1)

    tw = {
        "ewe": eb_we.T, "ews": eb_ws.T, "ewr": eb_wr.T, "ewu": eb_wu.T,
        "eb": eb_b.T,
        "nwagg": nb_wagg.T, "nwx": nb_wx.T, "nwu": nb_wu.T, "nb": nb_b.T,
        "gwe": gb_we.T, "gwn": gb_wn.T, "gwu": gb_wu.T, "gb": gb_b.T,
        "d1e": dec1_we.T, "d1s": dec1_ws.T, "d1r": dec1_wr.T,
        "d1u": dec1_wu.T, "d1b": dec1_b.T,
        "d2": dec2_w.T, "d2b": dec2_b.T,
    }

    te1 = _tile(E, 2048)
    g1 = E // te1
    ehT, ehTa, esumT = pl.pallas_call(
        _edge_encode_kernel,
        grid=(g1,),
        out_shape=(jax.ShapeDtypeStruct((EH2, E), jnp.float32),
                   jax.ShapeDtypeStruct((EH2A, E), jnp.bfloat16),
                   jax.ShapeDtypeStruct((EH2, 1), jnp.float32)),
        in_specs=[
            pl.BlockSpec((Fe, te1), lambda i: (0, i)),
            pl.BlockSpec((Fn, te1), lambda i: (0, i)),
            pl.BlockSpec((Fn, te1), lambda i: (0, i)),
            pl.BlockSpec((GH, 1), lambda i: (0, 0)),
            pl.BlockSpec((EH2, Fe), lambda i: (0, 0)),
            pl.BlockSpec((EH2, Fn), lambda i: (0, 0)),
            pl.BlockSpec((EH2, Fn), lambda i: (0, 0)),
            pl.BlockSpec((EH2, GH), lambda i: (0, 0)),
            pl.BlockSpec((EH2, 1), lambda i: (0, 0)),
        ],
        out_specs=(pl.BlockSpec((EH2, te1), lambda i: (0, i)),
                   pl.BlockSpec((EH2A, te1), lambda i: (0, i)),
                   pl.BlockSpec((EH2, 1), lambda i: (0, 0))),
        compiler_params=_CompilerParams(
            dimension_semantics=("arbitrary",),
            vmem_limit_bytes=_VMEM_LIMIT),
    )(eaT, xsT, xrT, uT, tw["ewe"], tw["ews"], tw["ewr"], tw["ewu"], tw["eb"])

    te2 = _tile(E, 256)
    g2 = E // te2
    tn = N // g2 if N % (g2 * g) == 0 else N
    if tn == N:
        g2 = 1
        te2 = E
    tnc = tn // g
    nrT, nsT, corrT, nsum2 = pl.pallas_call(
        _node_kernel,
        grid=(g2,),
        out_shape=(jax.ShapeDtypeStruct((NH2, E), jnp.float32),
                   jax.ShapeDtypeStruct((NH2, E), jnp.float32),
                   jax.ShapeDtypeStruct((NH2, 1), jnp.float32),
                   jax.ShapeDtypeStruct((1, g * NH2), jnp.float32)),
        in_specs=[
            pl.BlockSpec((tnc, g * Fn), lambda i: (i, 0)),
            pl.BlockSpec((g * Fn, g * NH2), lambda i: (0, 0)),
            pl.BlockSpec((E, 1), lambda i: (0, 0)),
            pl.BlockSpec((1, te2), lambda i: (0, i)),
            pl.BlockSpec((1, te2), lambda i: (0, i)),
            pl.BlockSpec((EH2A, E), lambda i: (0, 0)),
            pl.BlockSpec((Fn, te2), lambda i: (0, i)),
            pl.BlockSpec((Fn, te2), lambda i: (0, i)),
            pl.BlockSpec((GH, 1), lambda i: (0, 0)),
            pl.BlockSpec((NH2, EH2), lambda i: (0, 0)),
            pl.BlockSpec((NH2, Fn), lambda i: (0, 0)),
            pl.BlockSpec((NH2, GH), lambda i: (0, 0)),
            pl.BlockSpec((NH2, 1), lambda i: (0, 0)),
        ],
        out_specs=(pl.BlockSpec((NH2, te2), lambda i: (0, i)),
                   pl.BlockSpec((NH2, te2), lambda i: (0, i)),
                   pl.BlockSpec((NH2, 1), lambda i: (0, 0)),
                   pl.BlockSpec((1, g * NH2), lambda i: (0, 0))),
        scratch_shapes=[
            pltpu.VMEM((E, 128), jnp.int32),
        ],
        compiler_params=_CompilerParams(
            dimension_semantics=("arbitrary",),
            vmem_limit_bytes=_VMEM_LIMIT),
    )(xc, mk, rcol, rrow, srow, ehTa, xrT, xsT, uT,
      tw["nwagg"], tw["nwx"], tw["nwu"], tw["nb"])

    te4 = _tile(E, 2048)
    outT = pl.pallas_call(
        functools.partial(_decode_kernel, out_dim=OUT,
                          inv_e=1.0 / E, inv_n=1.0 / N),
        grid=(E // te4,),
        out_shape=jax.ShapeDtypeStruct((OUT, E), jnp.float32),
        in_specs=[
            pl.BlockSpec((EH2, te4), lambda i: (0, i)),
            pl.BlockSpec((NH2, te4), lambda i: (0, i)),
            pl.BlockSpec((NH2, te4), lambda i: (0, i)),
            pl.BlockSpec((Fn, te4), lambda i: (0, i)),
            pl.BlockSpec((Fn, te4), lambda i: (0, i)),
            pl.BlockSpec((EH2, 1), lambda i: (0, 0)),
            pl.BlockSpec((1, g * NH2), lambda i: (0, 0)),
            pl.BlockSpec((g * NH2, NH2), lambda i: (0, 0)),
            pl.BlockSpec((NH2, 1), lambda i: (0, 0)),
            pl.BlockSpec((GH, 1), lambda i: (0, 0)),
            pl.BlockSpec((GH2, EH2), lambda i: (0, 0)),
            pl.BlockSpec((GH2, NH2), lambda i: (0, 0)),
            pl.BlockSpec((GH2, GH), lambda i: (0, 0)),
            pl.BlockSpec((GH2, 1), lambda i: (0, 0)),
            pl.BlockSpec((EH2, EH2), lambda i: (0, 0)),
            pl.BlockSpec((EH2, NH2), lambda i: (0, 0)),
            pl.BlockSpec((EH2, NH2), lambda i: (0, 0)),
            pl.BlockSpec((EH2, GH2), lambda i: (0, 0)),
            pl.BlockSpec((EH2, 1), lambda i: (0, 0)),
            pl.BlockSpec((OUT2, EH2), lambda i: (0, 0)),
            pl.BlockSpec((OUT2, 1), lambda i: (0, 0)),
        ],
        out_specs=pl.BlockSpec((OUT, te4), lambda i: (0, i)),
        compiler_params=_CompilerParams(
            dimension_semantics=("arbitrary",),
            vmem_limit_bytes=_VMEM_LIMIT),
    )(ehT, nsT, nrT, xrT, xsT, esumT, nsum2, fk, corrT, uT,
      tw["gwe"], tw["gwn"], tw["gwu"], tw["gb"],
      tw["d1e"], tw["d1s"], tw["d1r"], tw["d1u"], tw["d1b"],
      tw["d2"], tw["d2b"])

    return outT.T


# speedup vs baseline: 3.5651x; 3.5651x over previous
import functools

import jax
import jax.numpy as jnp
from jax import lax
from jax.experimental import pallas as pl
from jax.experimental.pallas import tpu as pltpu

_CompilerParams = getattr(pltpu, "CompilerParams", None) or getattr(pltpu, "TPUCompilerParams")
_ANY_SPACE = getattr(pl, "ANY", None) or pltpu.TPUMemorySpace.ANY

_VMEM_LIMIT = 64 * 1024 * 1024


def _edge_encode_kernel(ea_ref, xs_ref, xr_ref, u_ref,
                        we_ref, ws_ref, wr_ref, wu_ref, b_ref,
                        ehT_ref, ehTa_ref, esum_ref):
    c = jnp.dot(wu_ref[...], u_ref[...], preferred_element_type=jnp.float32) + b_ref[...]
    y = (jnp.dot(we_ref[...], ea_ref[...], preferred_element_type=jnp.float32)
         + jnp.dot(ws_ref[...], xs_ref[...], preferred_element_type=jnp.float32)
         + jnp.dot(wr_ref[...], xr_ref[...], preferred_element_type=jnp.float32)
         + c)
    ehT = jnp.maximum(y, 0.0)
    ehT_ref[...] = ehT
    te = ehT.shape[1]
    ehTa_ref[...] = jnp.concatenate(
        [ehT.astype(jnp.bfloat16),
         jnp.ones((1, te), jnp.bfloat16),
         jnp.zeros((7, te), jnp.bfloat16)], axis=0)

    @pl.when(pl.program_id(0) == 0)
    def _():
        esum_ref[...] = jnp.zeros_like(esum_ref)

    esum_ref[...] += jnp.sum(ehT, axis=1, keepdims=True)


def _node_kernel(x_hbm, rcol_ref, rrow_ref, srow_ref, ehTa_ref,
                 xrT_ref, xsT_ref, u_ref,
                 wagg_ref, wx_ref, wu_ref, b_ref,
                 nrT_ref, nsT_ref, corr_ref, nsum_ref,
                 xbuf, dsem, rcb, *, tn, nsteps):
    i = pl.program_id(0)

    tq = tn // 4 if tn % 4 == 0 else tn

    def _start(step, slot):
        if tq != tn:
            for q in range(4):
                pltpu.make_async_copy(
                    x_hbm.at[pl.ds(step * tn + q * tq, tq), :],
                    xbuf.at[slot, pl.ds(q * tq, tq), :],
                    dsem.at[slot, q]).start()
        else:
            pltpu.make_async_copy(x_hbm.at[pl.ds(step * tn, tn), :],
                                  xbuf.at[slot], dsem.at[slot, 0]).start()

    def _wait(slot):
        if tq != tn:
            for q in range(4):
                sub = xbuf.at[slot, pl.ds(q * tq, tq), :]
                pltpu.make_async_copy(sub, sub, dsem.at[slot, q]).wait()
        else:
            pltpu.make_async_copy(xbuf.at[slot], xbuf.at[slot],
                                  dsem.at[slot, 0]).wait()

    @pl.when(i == 0)
    def _():
        rcb[...] = jnp.broadcast_to(rcol_ref[...], rcb.shape)
        _start(0, 0)
        if nsteps > 1:
            _start(1, 1)

    @pl.when(i + 2 < nsteps)
    def _():
        _start(i + 2, lax.rem(i + 2, 3))

    slot_i = lax.rem(i, 3)
    _wait(slot_i)

    c = jnp.dot(wu_ref[...], u_ref[...], preferred_element_type=jnp.float32) + b_ref[...]

    baseN = jnp.maximum(
        lax.dot_general(wx_ref[...], xbuf[slot_i], (((1,), (1,)), ((), ())),
                        preferred_element_type=jnp.float32) + c, 0.0)

    @pl.when(i == 0)
    def _():
        nsum_ref[...] = jnp.zeros_like(nsum_ref)
        corr_ref[...] = jnp.zeros_like(corr_ref)

    nsum_ref[...] += jnp.sum(baseN, axis=1, keepdims=True)

    rs = jnp.concatenate([rrow_ref[...], srow_ref[...]], axis=1)
    rcbv = rcb[...]
    nchunk = rs.shape[1] // 128
    mask = jnp.concatenate(
        [(rcbv == rs[:, k * 128:(k + 1) * 128]).astype(jnp.bfloat16)
         for k in range(nchunk)], axis=1)
    aggT2 = jnp.dot(ehTa_ref[...], mask, preferred_element_type=jnp.float32)

    te = rrow_ref.shape[1]
    aggrT = aggT2[:64, :te]
    multT = aggT2[64:65, :te]
    aggsT = aggT2[:64, te:]

    base_r = jnp.dot(wx_ref[...], xrT_ref[...], preferred_element_type=jnp.float32) + c
    base_s = jnp.dot(wx_ref[...], xsT_ref[...], preferred_element_type=jnp.float32) + c
    nrT = jnp.maximum(
        base_r + jnp.dot(wagg_ref[...], aggrT, preferred_element_type=jnp.float32), 0.0)
    nsT = jnp.maximum(
        base_s + jnp.dot(wagg_ref[...], aggsT, preferred_element_type=jnp.float32), 0.0)
    nrT_ref[...] = nrT
    nsT_ref[...] = nsT

    delta = (nrT - jnp.maximum(base_r, 0.0)) / multT
    corr_ref[...] += jnp.sum(delta, axis=1, keepdims=True)


def _decode_kernel(ehT_ref, nsT_ref, nrT_ref, xrT_ref, xsT_ref,
                   esum_ref, nsum_ref, corr_ref, u_ref,
                   gwe_ref, gwn_ref, gwu_ref, gb_ref,
                   w1e_ref, w1s_ref, w1r_ref, w1u_ref, b1_ref,
                   w2_ref, b2_ref, out_ref, *, out_dim, inv_e, inv_n):
    e_mean = esum_ref[...] * inv_e
    n_mean = (nsum_ref[...] + corr_ref[...]) * inv_n
    uy = (jnp.dot(gwe_ref[...], e_mean, preferred_element_type=jnp.float32)
          + jnp.dot(gwn_ref[...], n_mean, preferred_element_type=jnp.float32)
          + jnp.dot(gwu_ref[...], u_ref[...], preferred_element_type=jnp.float32)
          + gb_ref[...])
    uhT = jnp.maximum(uy, 0.0)

    cu = jnp.dot(w1u_ref[...], uhT, preferred_element_type=jnp.float32) + b1_ref[...]
    h = (jnp.dot(w1e_ref[...], ehT_ref[...], preferred_element_type=jnp.float32)
         + jnp.dot(w1s_ref[...], nsT_ref[...], preferred_element_type=jnp.float32)
         + jnp.dot(w1r_ref[...], nrT_ref[...], preferred_element_type=jnp.float32)
         + cu)
    h = jnp.maximum(h, 0.0)
    o = jnp.dot(w2_ref[...], h, preferred_element_type=jnp.float32) + b2_ref[...]
    o1 = o[:out_dim, :]
    o2 = o[out_dim:, :]
    xr_row = xrT_ref[2:3, :]
    xs_row = xsT_ref[2:3, :]
    out_ref[...] = o1 * (xr_row - o2 * xs_row)


def _tile(dim, tile):
    t = min(tile, dim)
    if dim % t != 0 or t % 128 != 0:
        t = dim
    return t


def kernel(eb_we, eb_ws, eb_wr, eb_wu, eb_b,
           nb_wagg, nb_wx, nb_wu, nb_b,
           gb_we, gb_wn, gb_wu, gb_b,
           dec1_we, dec1_ws, dec1_wr, dec1_wu, dec1_b,
           dec2_w, dec2_b,
           x, edge_index, edge_attr, u):
    sind, rind = edge_index[0], edge_index[1]
    N, Fn = x.shape
    E, Fe = edge_attr.shape
    GH = u.shape[1]
    EH2 = eb_b.shape[1]
    NH2 = nb_b.shape[1]
    GH2 = gb_b.shape[1]
    OUT2 = dec2_b.shape[1]
    OUT = OUT2 // 2
    EH2A = EH2 + 8

    xs = x[sind]
    xr = x[rind]
    eaT = edge_attr.T
    xsT = xs.T
    xrT = xr.T
    uT = u.T
    rind32 = rind.astype(jnp.int32)
    sind32 = sind.astype(jnp.int32)
    rrow = rind32.reshape(1, E)
    srow = sind32.reshape(1, E)
    rcol = rind32.reshape(E, 1)

    tw = {
        "ewe": eb_we.T, "ews": eb_ws.T, "ewr": eb_wr.T, "ewu": eb_wu.T,
        "eb": eb_b.T,
        "nwagg": nb_wagg.T, "nwx": nb_wx.T, "nwu": nb_wu.T, "nb": nb_b.T,
        "gwe": gb_we.T, "gwn": gb_wn.T, "gwu": gb_wu.T, "gb": gb_b.T,
        "d1e": dec1_we.T, "d1s": dec1_ws.T, "d1r": dec1_wr.T,
        "d1u": dec1_wu.T, "d1b": dec1_b.T,
        "d2": dec2_w.T, "d2b": dec2_b.T,
    }

    te1 = _tile(E, 2048)
    g1 = E // te1
    ehT, ehTa, esumT = pl.pallas_call(
        _edge_encode_kernel,
        grid=(g1,),
        out_shape=(jax.ShapeDtypeStruct((EH2, E), jnp.float32),
                   jax.ShapeDtypeStruct((EH2A, E), jnp.bfloat16),
                   jax.ShapeDtypeStruct((EH2, 1), jnp.float32)),
        in_specs=[
            pl.BlockSpec((Fe, te1), lambda i: (0, i)),
            pl.BlockSpec((Fn, te1), lambda i: (0, i)),
            pl.BlockSpec((Fn, te1), lambda i: (0, i)),
            pl.BlockSpec((GH, 1), lambda i: (0, 0)),
            pl.BlockSpec((EH2, Fe), lambda i: (0, 0)),
            pl.BlockSpec((EH2, Fn), lambda i: (0, 0)),
            pl.BlockSpec((EH2, Fn), lambda i: (0, 0)),
            pl.BlockSpec((EH2, GH), lambda i: (0, 0)),
            pl.BlockSpec((EH2, 1), lambda i: (0, 0)),
        ],
        out_specs=(pl.BlockSpec((EH2, te1), lambda i: (0, i)),
                   pl.BlockSpec((EH2A, te1), lambda i: (0, i)),
                   pl.BlockSpec((EH2, 1), lambda i: (0, 0))),
        compiler_params=_CompilerParams(
            dimension_semantics=("arbitrary",),
            vmem_limit_bytes=_VMEM_LIMIT),
    )(eaT, xsT, xrT, uT, tw["ewe"], tw["ews"], tw["ewr"], tw["ewu"], tw["eb"])

    te2 = _tile(E, 256)
    g2 = E // te2
    tn = N // g2 if N % g2 == 0 else N
    if tn == N:
        g2 = 1
        te2 = E
    nrT, nsT, corrT, nsumT = pl.pallas_call(
        functools.partial(_node_kernel, tn=tn, nsteps=g2),
        grid=(g2,),
        out_shape=(jax.ShapeDtypeStruct((NH2, E), jnp.float32),
                   jax.ShapeDtypeStruct((NH2, E), jnp.float32),
                   jax.ShapeDtypeStruct((NH2, 1), jnp.float32),
                   jax.ShapeDtypeStruct((NH2, 1), jnp.float32)),
        in_specs=[
            pl.BlockSpec(memory_space=_ANY_SPACE),
            pl.BlockSpec((E, 1), lambda i: (0, 0)),
            pl.BlockSpec((1, te2), lambda i: (0, i)),
            pl.BlockSpec((1, te2), lambda i: (0, i)),
            pl.BlockSpec((EH2A, E), lambda i: (0, 0)),
            pl.BlockSpec((Fn, te2), lambda i: (0, i)),
            pl.BlockSpec((Fn, te2), lambda i: (0, i)),
            pl.BlockSpec((GH, 1), lambda i: (0, 0)),
            pl.BlockSpec((NH2, EH2), lambda i: (0, 0)),
            pl.BlockSpec((NH2, Fn), lambda i: (0, 0)),
            pl.BlockSpec((NH2, GH), lambda i: (0, 0)),
            pl.BlockSpec((NH2, 1), lambda i: (0, 0)),
        ],
        out_specs=(pl.BlockSpec((NH2, te2), lambda i: (0, i)),
                   pl.BlockSpec((NH2, te2), lambda i: (0, i)),
                   pl.BlockSpec((NH2, 1), lambda i: (0, 0)),
                   pl.BlockSpec((NH2, 1), lambda i: (0, 0))),
        scratch_shapes=[
            pltpu.VMEM((3, tn, Fn), jnp.float32),
            pltpu.SemaphoreType.DMA((3, 4)),
            pltpu.VMEM((E, 128), jnp.int32),
        ],
        compiler_params=_CompilerParams(
            dimension_semantics=("arbitrary",),
            vmem_limit_bytes=_VMEM_LIMIT),
    )(x, rcol, rrow, srow, ehTa, xrT, xsT, uT,
      tw["nwagg"], tw["nwx"], tw["nwu"], tw["nb"])

    te4 = _tile(E, 2048)
    outT = pl.pallas_call(
        functools.partial(_decode_kernel, out_dim=OUT,
                          inv_e=1.0 / E, inv_n=1.0 / N),
        grid=(E // te4,),
        out_shape=jax.ShapeDtypeStruct((OUT, E), jnp.float32),
        in_specs=[
            pl.BlockSpec((EH2, te4), lambda i: (0, i)),
            pl.BlockSpec((NH2, te4), lambda i: (0, i)),
            pl.BlockSpec((NH2, te4), lambda i: (0, i)),
            pl.BlockSpec((Fn, te4), lambda i: (0, i)),
            pl.BlockSpec((Fn, te4), lambda i: (0, i)),
            pl.BlockSpec((EH2, 1), lambda i: (0, 0)),
            pl.BlockSpec((NH2, 1), lambda i: (0, 0)),
            pl.BlockSpec((NH2, 1), lambda i: (0, 0)),
            pl.BlockSpec((GH, 1), lambda i: (0, 0)),
            pl.BlockSpec((GH2, EH2), lambda i: (0, 0)),
            pl.BlockSpec((GH2, NH2), lambda i: (0, 0)),
            pl.BlockSpec((GH2, GH), lambda i: (0, 0)),
            pl.BlockSpec((GH2, 1), lambda i: (0, 0)),
            pl.BlockSpec((EH2, EH2), lambda i: (0, 0)),
            pl.BlockSpec((EH2, NH2), lambda i: (0, 0)),
            pl.BlockSpec((EH2, NH2), lambda i: (0, 0)),
            pl.BlockSpec((EH2, GH2), lambda i: (0, 0)),
            pl.BlockSpec((EH2, 1), lambda i: (0, 0)),
            pl.BlockSpec((OUT2, EH2), lambda i: (0, 0)),
            pl.BlockSpec((OUT2, 1), lambda i: (0, 0)),
        ],
        out_specs=pl.BlockSpec((OUT, te4), lambda i: (0, i)),
        compiler_params=_CompilerParams(
            dimension_semantics=("arbitrary",),
            vmem_limit_bytes=_VMEM_LIMIT),
    )(ehT, nsT, nrT, xrT, xsT, esumT, nsumT, corrT, uT,
      tw["gwe"], tw["gwn"], tw["gwu"], tw["gb"],
      tw["d1e"], tw["d1s"], tw["d1r"], tw["d1u"], tw["d1b"],
      tw["d2"], tw["d2b"])

    return outT.T
